# Initial kernel scaffold; baseline (speedup 1.0000x reference)
import jax
import jax.numpy as jnp
from jax.experimental import pallas as pl


def kernel(hidden_states, gate_w, w1, w2):
    raise NotImplementedError("write your pallas kernel here")


# speedup vs baseline: 2.7010x; 2.7010x over previous
import functools

import jax
import jax.numpy as jnp
from jax.experimental import pallas as pl
from jax.experimental.pallas import tpu as pltpu

TOPK = 2


def _gelu_exact(x):
    return 0.5 * x * (1.0 + jax.lax.erf(x * 0.7071067811865476))


def _router_kernel(x_ref, gw_ref, logits_ref, wfull_ref):
    x = x_ref[...]
    gw = gw_ref[...]
    logits = jnp.dot(x, gw, preferred_element_type=jnp.float32)
    logits_ref[...] = logits
    E = logits.shape[1]
    col = jax.lax.broadcasted_iota(jnp.int32, logits.shape, 1)
    m1 = jnp.max(logits, axis=1, keepdims=True)
    i1 = jnp.min(jnp.where(logits == m1, col, E), axis=1, keepdims=True)
    masked = jnp.where(col == i1, -jnp.inf, logits)
    m2 = jnp.max(masked, axis=1, keepdims=True)
    i2 = jnp.min(jnp.where(masked == m2, col, E), axis=1, keepdims=True)
    e2 = jnp.exp(m2 - m1)
    denom = 1.0 + e2
    p1 = 1.0 / denom
    p2 = e2 / denom
    wfull_ref[...] = (jnp.where(col == i1, p1, 0.0)
                      + jnp.where(col == i2, p2, 0.0))


def _ffn_kernel(wfull_ref, x_ref, w1_ref, w2_ref, out_ref):
    e = pl.program_id(0)
    tt = pl.program_id(1)
    TT = x_ref.shape[0] // pl.num_programs(1)
    xs = x_ref[pl.ds(tt * TT, TT), :]
    xb = xs.astype(jnp.bfloat16)
    h = jnp.dot(xb, w1_ref[0], preferred_element_type=jnp.float32)
    h = _gelu_exact(h)
    o = jnp.dot(h.astype(jnp.bfloat16), w2_ref[0],
                preferred_element_type=jnp.float32)
    wf = wfull_ref[pl.ds(tt * TT, TT), :]
    colE = jax.lax.broadcasted_iota(jnp.int32, wf.shape, 1)
    scale = jnp.sum(jnp.where(colE == e, wf, 0.0), axis=1, keepdims=True)
    contrib = scale * o

    @pl.when(e == 0)
    def _init():
        out_ref[pl.ds(tt * TT, TT), :] = contrib

    @pl.when(e != 0)
    def _acc():
        out_ref[pl.ds(tt * TT, TT), :] += contrib


@jax.jit
def kernel(hidden_states, gate_w, w1, w2):
    B, S, H = hidden_states.shape
    E = gate_w.shape[1]
    I = w1.shape[2]
    T = B * S
    flat = hidden_states.reshape(T, H)

    logits, wfull = pl.pallas_call(
        _router_kernel,
        out_shape=(
            jax.ShapeDtypeStruct((T, E), jnp.float32),
            jax.ShapeDtypeStruct((T, E), jnp.float32),
        ),
    )(flat, gate_w)

    w1b = w1.astype(jnp.bfloat16)
    w2b = w2.astype(jnp.bfloat16)

    TT = 256
    n_tt = T // TT
    out = pl.pallas_call(
        _ffn_kernel,
        grid=(E, n_tt),
        in_specs=[
            pl.BlockSpec((T, E), lambda e, t: (0, 0)),
            pl.BlockSpec((T, H), lambda e, t: (0, 0)),
            pl.BlockSpec((1, H, I), lambda e, t: (e, 0, 0)),
            pl.BlockSpec((1, I, H), lambda e, t: (e, 0, 0)),
        ],
        out_specs=pl.BlockSpec((T, H), lambda e, t: (0, 0)),
        out_shape=jax.ShapeDtypeStruct((T, H), jnp.float32),
    )(wfull, flat, w1b, w2b)

    return out.reshape(B, S, H), logits.reshape(B, S, E)
